# Initial kernel scaffold; baseline (speedup 1.0000x reference)
import jax
import jax.numpy as jnp
from jax.experimental import pallas as pl


def kernel(x, edge_index, batch, W1, b1, W2, b2, Wfc, bfc):
    raise NotImplementedError("write your pallas kernel here")


# speedup vs baseline: 18.1388x; 18.1388x over previous
import functools

import jax
import jax.numpy as jnp
from jax import lax
from jax.experimental import pallas as pl
from jax.experimental.pallas import tpu as pltpu
from jax.experimental.pallas import tpu_sc as plsc

NC = 2
NS = 16
L = 16

N = 10000
E = 320000
D = 128
G = 16
EPS = E // NS
WB = 80
PAD = 176
CAP = EPS + PAD
NHALF = N // NC
ACC_ROWS = 5120
DEG_WIN = EPS // WB // NC


def _stage_window(stage2d, b, src1d, off):
    for k in range(WB // L):
        stage2d[b, pl.ds(k * L, L)] = src1d[pl.ds(off + k * L, L)]


def _sc_route_kernel(src_h, dst_h, ones_h, zeros1, routed_src, routed_dst,
                     counts, deg_out,
                     src_v, dst_v, osrc_v, odst_v, ones_v, cnt_v, stage,
                     bounce1, deg_acc, sem):
    del sem
    c = lax.axis_index("c")
    s = lax.axis_index("s")
    lo = c * NHALF

    @pl.when(s < 10)
    def _():
        pltpu.sync_copy(zeros1.at[pl.ds(s * 1000, 1000)], bounce1)
        pltpu.sync_copy(bounce1, deg_acc.at[pl.ds(s * 1000, 1000)])

    pltpu.sync_copy(ones_h, ones_v)
    pltpu.sync_copy(src_h.at[pl.ds(s * EPS, EPS)], src_v)
    pltpu.sync_copy(dst_h.at[pl.ds(s * EPS, EPS)], dst_v)
    plsc.subcore_barrier()

    @pl.loop(0, DEG_WIN)
    def _(j):
        w = c * DEG_WIN + j
        _stage_window(stage, 0, dst_v, w * WB)
        pltpu.sync_copy(ones_v.at[pl.ds(0, WB)],
                        deg_acc.at[stage.at[0]], add=True)

    def body(i, cnt):
        sl = pl.ds(i * L, L)
        d = dst_v[sl]
        keep = (d >= lo) & (d < lo + NHALF)
        incl = plsc.cumsum(keep.astype(jnp.int32))
        pos = cnt + incl - 1
        plsc.store_scatter(odst_v, [pos], d - lo, mask=keep)
        plsc.store_scatter(osrc_v, [pos], src_v[sl], mask=keep)
        return cnt + incl[L - 1]

    cnt = pl.loop(0, EPS // L, init_carry=jnp.int32(0))(body)

    iota = lax.iota(jnp.int32, L)
    trash = NHALF + iota * 2
    zero16 = jnp.zeros((L,), jnp.int32)
    full = iota >= 0
    for k in range(PAD // L):
        pos = cnt + k * L + iota
        plsc.store_scatter(odst_v, [pos], trash, mask=full)
        plsc.store_scatter(osrc_v, [pos], zero16, mask=full)

    w32 = c * NS + s
    pltpu.sync_copy(osrc_v, routed_src.at[pl.ds(w32 * CAP, CAP)])
    pltpu.sync_copy(odst_v, routed_dst.at[pl.ds(w32 * CAP, CAP)])
    cnt_v[...] = jnp.broadcast_to(cnt, (L,))
    pltpu.sync_copy(cnt_v, counts.at[pl.ds(w32 * L, L)])

    plsc.subcore_barrier()

    @pl.when(s < 10)
    def _():
        pltpu.sync_copy(deg_acc.at[pl.ds(s * 1000, 1000)], bounce1)
        pltpu.sync_copy(bounce1, deg_out.at[pl.ds(c * N + s * 1000, 1000)])


def _sc_aggregate_kernel(p_hbm, routed_src, routed_dst, counts, zeros2, out,
                         src_v, dst_v, buf0, buf1, cnt_v, stage, bounce,
                         acc, sem0, sem1):
    c = lax.axis_index("c")
    s = lax.axis_index("s")
    w32 = c * NS + s

    @pl.loop(0, 2)
    def _(j):
        off = s * 320 + j * 160
        pltpu.sync_copy(zeros2.at[pl.ds(off, 160)], bounce.at[pl.ds(0, 160)])
        pltpu.sync_copy(bounce.at[pl.ds(0, 160)], acc.at[pl.ds(off, 160)])

    pltpu.sync_copy(routed_src.at[pl.ds(w32 * CAP, CAP)], src_v)
    pltpu.sync_copy(routed_dst.at[pl.ds(w32 * CAP, CAP)], dst_v)
    pltpu.sync_copy(counts.at[pl.ds(w32 * L, L)], cnt_v)
    plsc.subcore_barrier()

    cnt = cnt_v[...][0]
    nwin = 2 * jnp.maximum((cnt + 2 * WB - 1) // (2 * WB), 1)

    pltpu.async_copy(p_hbm.at[src_v.at[pl.ds(0, WB)]], buf0, sem0)
    pltpu.async_copy(p_hbm.at[src_v.at[pl.ds(WB, WB)]], buf1, sem1)

    @pl.loop(0, nwin, step=2)
    def _(j):
        for b, (buf, sem) in enumerate(((buf0, sem0), (buf1, sem1))):
            wv = j + b
            pltpu.make_async_copy(
                p_hbm.at[src_v.at[pl.ds(wv * WB, WB)]], buf, sem).wait()
            _stage_window(stage, b, dst_v, wv * WB)
            pltpu.sync_copy(buf, acc.at[stage.at[b]], add=True)

            @pl.when(wv + 2 < nwin)
            def _():
                pltpu.async_copy(
                    p_hbm.at[src_v.at[pl.ds((wv + 2) * WB, WB)]], buf, sem)

    plsc.subcore_barrier()

    @pl.when(s < 5)
    def _():
        @pl.loop(0, 5)
        def _(j):
            off = s * 1000 + j * 200
            pltpu.sync_copy(acc.at[pl.ds(off, 200)], bounce)
            pltpu.sync_copy(bounce, out.at[pl.ds(c * NHALF + off, 200)])


_SC_MESH = plsc.VectorSubcoreMesh(core_axis_name="c", subcore_axis_name="s")
_SC_PARAMS = pltpu.CompilerParams(needs_layout_passes=False)

_sc_route = functools.partial(
    pl.kernel,
    out_type=(
        jax.ShapeDtypeStruct((NC * NS * CAP,), jnp.int32),
        jax.ShapeDtypeStruct((NC * NS * CAP,), jnp.int32),
        jax.ShapeDtypeStruct((NC * NS * L,), jnp.int32),
        jax.ShapeDtypeStruct((NC * N,), jnp.float32),
    ),
    mesh=_SC_MESH,
    compiler_params=_SC_PARAMS,
    scratch_types=[
        pltpu.VMEM((EPS,), jnp.int32),
        pltpu.VMEM((EPS,), jnp.int32),
        pltpu.VMEM((CAP,), jnp.int32),
        pltpu.VMEM((CAP,), jnp.int32),
        pltpu.VMEM((D,), jnp.float32),
        pltpu.VMEM((L,), jnp.int32),
        pltpu.VMEM((2, WB), jnp.int32),
        pltpu.VMEM((1000,), jnp.float32),
        pltpu.VMEM_SHARED((N,), jnp.float32),
        pltpu.SemaphoreType.DMA,
    ],
)(_sc_route_kernel)

_sc_aggregate = functools.partial(
    pl.kernel,
    out_type=jax.ShapeDtypeStruct((N, D), jnp.float32),
    mesh=_SC_MESH,
    compiler_params=_SC_PARAMS,
    scratch_types=[
        pltpu.VMEM((CAP,), jnp.int32),
        pltpu.VMEM((CAP,), jnp.int32),
        pltpu.VMEM((WB, D), jnp.float32),
        pltpu.VMEM((WB, D), jnp.float32),
        pltpu.VMEM((L,), jnp.int32),
        pltpu.VMEM((2, WB), jnp.int32),
        pltpu.VMEM((200, D), jnp.float32),
        pltpu.VMEM_SHARED((ACC_ROWS, D), jnp.float32),
        pltpu.SemaphoreType.DMA,
        pltpu.SemaphoreType.DMA,
    ],
)(_sc_aggregate_kernel)


BN = 1000


def _tc_scale_matmul_kernel(x_ref, w_ref, dinv_ref, o_ref):
    h = lax.dot_general(x_ref[...], w_ref[...], (((1,), (1,)), ((), ())),
                        preferred_element_type=jnp.float32)
    o_ref[...] = h * dinv_ref[...]


def _tc_layer_kernel(s_ref, p_ref, dinv_ref, b_ref, w_ref, o_ref):
    agg = (s_ref[...] + p_ref[...]) * dinv_ref[...] + b_ref[...]
    h = jnp.maximum(agg, 0.0)
    o_ref[...] = lax.dot_general(h, w_ref[...], (((1,), (1,)), ((), ())),
                                 preferred_element_type=jnp.float32) * dinv_ref[...]


def _tc_pool_kernel(s_ref, p_ref, dinv_ref, b_ref, oneh_ref, wfc_ref,
                    bfc_ref, o_ref, pool_acc, cnt_acc):
    k = pl.program_id(0)

    @pl.when(k == 0)
    def _():
        pool_acc[...] = jnp.zeros_like(pool_acc)
        cnt_acc[...] = jnp.zeros_like(cnt_acc)

    agg = (s_ref[...] + p_ref[...]) * dinv_ref[...] + b_ref[...]
    h = jnp.maximum(agg, 0.0)
    oneh = oneh_ref[...]
    pool_acc[...] += lax.dot_general(oneh, h, (((0,), (0,)), ((), ())),
                                     preferred_element_type=jnp.float32, precision=lax.Precision.HIGHEST)
    cnt_acc[...] += lax.dot_general(oneh, jnp.ones_like(h),
                                    (((0,), (0,)), ((), ())),
                                    preferred_element_type=jnp.float32, precision=lax.Precision.HIGHEST)

    @pl.when(k == pl.num_programs(0) - 1)
    def _():
        pooled = pool_acc[...] / jnp.maximum(cnt_acc[...], 1.0)
        o_ref[...] = jnp.sum(pooled * wfc_ref[...], axis=1,
                             keepdims=True) + bfc_ref[0, 0]


def _tc_scale_matmul(x, w, dinv_col):
    row = pl.BlockSpec((BN, D), lambda i: (i, 0))
    return pl.pallas_call(
        _tc_scale_matmul_kernel,
        grid=(N // BN,),
        in_specs=[
            row,
            pl.BlockSpec((D, D), lambda i: (0, 0)),
            pl.BlockSpec((BN, 1), lambda i: (i, 0)),
        ],
        out_specs=row,
        out_shape=jax.ShapeDtypeStruct((N, D), jnp.float32),
    )(x, w, dinv_col)


def _tc_layer(s, p, dinv_col, b_row, w):
    row = pl.BlockSpec((BN, D), lambda i: (i, 0))
    return pl.pallas_call(
        _tc_layer_kernel,
        grid=(N // BN,),
        in_specs=[
            row, row,
            pl.BlockSpec((BN, 1), lambda i: (i, 0)),
            pl.BlockSpec((1, D), lambda i: (0, 0)),
            pl.BlockSpec((D, D), lambda i: (0, 0)),
        ],
        out_specs=row,
        out_shape=jax.ShapeDtypeStruct((N, D), jnp.float32),
    )(s, p, dinv_col, b_row, w)


def _tc_pool(s, p, dinv_col, b_row, oneh, wfc, bfc_2d):
    row = pl.BlockSpec((BN, D), lambda i: (i, 0))
    return pl.pallas_call(
        _tc_pool_kernel,
        grid=(N // BN,),
        in_specs=[
            row, row,
            pl.BlockSpec((BN, 1), lambda i: (i, 0)),
            pl.BlockSpec((1, D), lambda i: (0, 0)),
            pl.BlockSpec((BN, G), lambda i: (i, 0)),
            pl.BlockSpec((1, D), lambda i: (0, 0)),
            pl.BlockSpec((1, 1), lambda i: (0, 0)),
        ],
        out_specs=pl.BlockSpec((G, 1), lambda i: (0, 0)),
        out_shape=jax.ShapeDtypeStruct((G, 1), jnp.float32),
        scratch_shapes=[
            pltpu.VMEM((G, D), jnp.float32),
            pltpu.VMEM((G, D), jnp.float32),
        ],
    )(s, p, dinv_col, b_row, oneh, wfc, bfc_2d)


@jax.jit
def kernel(x, edge_index, batch, W1, b1, W2, b2, Wfc, bfc):
    src_h = edge_index[0]
    dst_h = edge_index[1]
    zeros1 = jnp.zeros((N,), jnp.float32)
    zeros2 = jnp.zeros((ACC_ROWS, D), jnp.float32)
    ones_h = jnp.ones((D,), jnp.float32)

    rsrc, rdst, counts, deg_parts = _sc_route(src_h, dst_h, ones_h, zeros1)
    deg = deg_parts[:N] + deg_parts[N:] + 1.0
    dinv_col = (1.0 / jnp.sqrt(deg))[:, None]

    b1_row = b1.reshape(1, D)
    b2_row = b2.reshape(1, D)
    oneh = (batch[:, None] == jnp.arange(G, dtype=batch.dtype)[None, :])
    oneh = oneh.astype(jnp.float32)

    p1 = _tc_scale_matmul(x, W1, dinv_col)
    s1 = _sc_aggregate(p1, rsrc, rdst, counts, zeros2)
    p2 = _tc_layer(s1, p1, dinv_col, b1_row, W2)
    s2 = _sc_aggregate(p2, rsrc, rdst, counts, zeros2)
    return _tc_pool(s2, p2, dinv_col, b2_row, oneh, Wfc, bfc.reshape(1, 1))
